# Initial kernel scaffold; baseline (speedup 1.0000x reference)
import jax
import jax.numpy as jnp
from jax.experimental import pallas as pl


def kernel(queries, candidates):
    raise NotImplementedError("write your pallas kernel here")


# speedup vs baseline: 16.0469x; 16.0469x over previous
import jax, jax.numpy as jnp
import numpy as np
from jax import lax
from jax.experimental import pallas as pl
from jax.experimental.pallas import tpu as pltpu

K_TOPK = 14
NQ, D = 1024, 16
N_REAL = 1_000_000
B1 = 16384
NBLK = 62
N_PAD = NBLK * B1
SUB = 2048
CHUNK = 128
NCHUNK = N_PAD // CHUNK
SEL_W = 16
RS = K_TOPK * CHUNK
NEG = np.float32(-np.inf)
BIGI = np.int32(2**30)


def _stage1(q_ref, c_ref, m_ref):
    g = pl.program_id(0)
    q = q_ref[...]
    for s in range(B1 // SUB):
        c = c_ref[:, s * SUB:(s + 1) * SUB]
        scores = lax.dot_general(q, c, (((1,), (0,)), ((), ())),
                                 preferred_element_type=jnp.float32)
        col = g * B1 + s * SUB + lax.broadcasted_iota(jnp.int32, (NQ, SUB), 1)
        scores = jnp.where(col < N_REAL, scores, NEG)
        sr = scores.reshape(NQ, SUB // CHUNK, CHUNK)
        m_ref[:, s * (SUB // CHUNK):(s + 1) * (SUB // CHUNK)] = jnp.max(sr, axis=2)


S2_BLOCKS = [(lo, min(1024, NCHUNK - lo)) for lo in range(0, NCHUNK, 1024)]


def _stage2(m_ref, sel_ref, wrk_ref):
    vparts, iparts = [], []
    for lo, w in S2_BLOCKS:
        wrk_ref[:, :w] = m_ref[:, lo:lo + w]
        cidx = lo + lax.broadcasted_iota(jnp.int32, (NQ, w), 1)
        for _ in range(K_TOPK):
            v = wrk_ref[:, :w]
            mx = jnp.max(v, axis=1, keepdims=True)
            ism = v == mx
            pick = jnp.min(jnp.where(ism, cidx, BIGI), axis=1, keepdims=True)
            vparts.append(mx)
            iparts.append(pick)
            wrk_ref[:, :w] = jnp.where(cidx == pick, NEG, v)
    av = jnp.concatenate(vparts, axis=1)
    ai = jnp.concatenate(iparts, axis=1)
    cols = []
    for _ in range(K_TOPK):
        mx = jnp.max(av, axis=1, keepdims=True)
        ism = av == mx
        pick = jnp.min(jnp.where(ism, ai, BIGI), axis=1, keepdims=True)
        cols.append(pick)
        av = jnp.where(ism & (ai == pick), NEG, av)
    cols.append(jnp.zeros((NQ, SEL_W - K_TOPK), jnp.int32))
    sel_ref[...] = jnp.concatenate(cols, axis=1)


def _stage4(v_ref, g_ref, ov_ref, oi_ref, wrk_ref):
    gi = g_ref[...]
    wrk_ref[...] = jnp.where(gi < N_REAL, v_ref[...], NEG)
    vcols, icols = [], []
    for _ in range(K_TOPK):
        v = wrk_ref[...]
        mx = jnp.max(v, axis=1, keepdims=True)
        ism = v == mx
        pick = jnp.min(jnp.where(ism, gi, BIGI), axis=1, keepdims=True)
        vcols.append(mx)
        icols.append(pick)
        wrk_ref[...] = jnp.where(ism & (gi == pick), NEG, v)
    ov_ref[...] = jnp.concatenate(vcols, axis=1)
    oi_ref[...] = jnp.concatenate(icols, axis=1)


def kernel(queries, candidates):
    c_bf = jnp.pad(candidates, ((0, N_PAD - N_REAL), (0, 0))).astype(jnp.bfloat16)
    q_bf = queries.astype(jnp.bfloat16)
    c_t = c_bf.T
    m = pl.pallas_call(
        _stage1,
        grid=(NBLK,),
        in_specs=[pl.BlockSpec((NQ, D), lambda g: (0, 0)),
                  pl.BlockSpec((D, B1), lambda g: (0, g))],
        out_specs=pl.BlockSpec((NQ, CHUNK), lambda g: (0, g)),
        out_shape=jax.ShapeDtypeStruct((NQ, NCHUNK), jnp.float32),
    )(q_bf, c_t)
    sel = pl.pallas_call(
        _stage2,
        out_shape=jax.ShapeDtypeStruct((NQ, SEL_W), jnp.int32),
        scratch_shapes=[pltpu.VMEM((NQ, 1024), jnp.float32)],
    )(m)
    ctab = c_bf.astype(jnp.float32).reshape(NCHUNK, CHUNK, D)
    sel14 = sel[:, :K_TOPK]
    chunks = ctab[sel14]
    vals = jnp.einsum('qd,qkcd->qkc', q_bf.astype(jnp.float32),
                      chunks).reshape(NQ, RS)
    gidx = (sel14[:, :, None] * CHUNK +
            jnp.arange(CHUNK)[None, None, :]).reshape(NQ, RS).astype(jnp.int32)
    values, indices = pl.pallas_call(
        _stage4,
        out_shape=[jax.ShapeDtypeStruct((NQ, K_TOPK), jnp.float32),
                   jax.ShapeDtypeStruct((NQ, K_TOPK), jnp.int32)],
        scratch_shapes=[pltpu.VMEM((NQ, RS), jnp.float32)],
    )(vals, gidx)
    return (values, indices)
